# Initial kernel scaffold; baseline (speedup 1.0000x reference)
import jax
import jax.numpy as jnp
from jax.experimental import pallas as pl


def kernel(neighbor_index, vertices, directions):
    raise NotImplementedError("write your pallas kernel here")


# speedup vs baseline: 16.5613x; 16.5613x over previous
import functools

import jax
import jax.numpy as jnp
from jax import lax
from jax.experimental import pallas as pl
from jax.experimental.pallas import tpu as pltpu
from jax.experimental.pallas import tpu_sc as plsc

_NB = 16
_NVB = 80


def _sc_gather(table, idx_w, n_rows):
    info = plsc.get_sparse_core_info()
    nw = info.num_cores * info.num_subcores
    per_w = idx_w.shape[1]
    n_vec = per_w // 16

    mesh = plsc.VectorSubcoreMesh(core_axis_name="c", subcore_axis_name="s")

    @functools.partial(
        pl.kernel,
        mesh=mesh,
        out_type=jax.ShapeDtypeStruct((nw, per_w * 3), jnp.float32),
        scratch_types=[
            pltpu.VMEM((table.shape[0] * 3,), jnp.float32),
            pltpu.VMEM((per_w,), jnp.int32),
            pltpu.VMEM((per_w * 3,), jnp.float32),
        ],
        compiler_params=pltpu.CompilerParams(needs_layout_passes=False),
    )
    def gather_k(table_hbm, idx_hbm, out_hbm, table_v, idx_v, rows_v):
        wid = lax.axis_index("s") * info.num_cores + lax.axis_index("c")
        pltpu.sync_copy(table_hbm, table_v)
        pltpu.sync_copy(idx_hbm.at[wid], idx_v)

        def body(i, carry):
            base = pl.multiple_of(i * 16, 16)
            iv = idx_v[pl.ds(base, 16)] * 3
            obase = pl.multiple_of(i * 48, 16)
            for c in range(3):
                vals = plsc.load_gather(table_v, [iv + c])
                rows_v[pl.ds(obase + c * 16, 16)] = vals
            return carry

        lax.fori_loop(0, n_vec, body, 0)
        pltpu.sync_copy(rows_v, out_hbm.at[wid])

    return gather_k(table.reshape(-1), idx_w)


def _conv_body(s_ref, g_ref, c_ref, o_ref):
    s = s_ref[...]
    s2 = jnp.sum(s * s, axis=0, keepdims=True)
    sn = s * (1.0 / jnp.maximum(jnp.sqrt(s2), 1e-12))

    g = g_ref[...]
    c = c_ref[...]
    dx = g[:, 0, :] - c[:, 0:1]
    dy = g[:, 1, :] - c[:, 1:2]
    dz = g[:, 2, :] - c[:, 2:3]
    n2 = dx * dx + dy * dy + dz * dz
    inv = 1.0 / jnp.maximum(jnp.sqrt(n2), 1e-12)
    th = ((dx * inv)[:, :, None] * sn[0:1, :].reshape(1, 1, 256)
          + (dy * inv)[:, :, None] * sn[1:2, :].reshape(1, 1, 256)
          + (dz * inv)[:, :, None] * sn[2:3, :].reshape(1, 1, 256))
    th = jnp.maximum(th, 0.0)
    m = jnp.max(th, axis=1)
    o_ref[...] = m[:, 0:64] + m[:, 64:128] + m[:, 128:192] + m[:, 192:256]


def _tc_conv(directions, g3, table, n_rows):
    grid = (n_rows // _NVB,)
    return pl.pallas_call(
        _conv_body,
        grid=grid,
        in_specs=[
            pl.BlockSpec((3, 256), lambda i: (0, 0)),
            pl.BlockSpec((_NVB, 3, _NB), lambda i: (i, 0, 0)),
            pl.BlockSpec((_NVB, 3), lambda i: (i, 0)),
        ],
        out_specs=pl.BlockSpec((_NVB, 64), lambda i: (i, 0)),
        out_shape=jax.ShapeDtypeStruct((n_rows, 64), jnp.float32),
        compiler_params=pltpu.CompilerParams(
            dimension_semantics=("parallel",),
        ),
    )(directions, g3, table)


def kernel(neighbor_index, vertices, directions):
    bs, v, nb = neighbor_index.shape
    n_rows = bs * v
    n_idx = bs * v * nb
    nw = 32
    table = vertices.reshape(n_rows, 3)
    idx_w = (
        neighbor_index + (jnp.arange(bs, dtype=jnp.int32) * v)[:, None, None]
    ).reshape(nw, n_idx // nw)
    gathered = _sc_gather(table, idx_w, n_idx)
    g3 = gathered.reshape(n_rows, 3, nb)
    out = _tc_conv(directions, g3, table, n_rows)
    return out.reshape(bs, v, 64)
